# speedup vs baseline: 1.6891x; 1.6891x over previous
import functools

import jax
import jax.numpy as jnp
from jax import lax
from jax.experimental import pallas as pl
from jax.experimental.pallas import tpu as pltpu
from jax.experimental.pallas import tpu_sc as plsc

_B = 16384
_D = 4096
_NW = 32
_BPW = _B // _NW
_CH = 16
_NCHUNK = _BPW // _CH


def _embed_kernel(x_hbm, table_hbm, out_hbm, idx_v, rows_v, sem):
    nc = 2
    wid = lax.axis_index("s") * nc + lax.axis_index("c")
    base = wid * _BPW
    pltpu.sync_copy(x_hbm.at[pl.ds(base, _BPW)], idx_v)

    def body(c, _):
        pltpu.async_copy(
            table_hbm.at[idx_v.at[pl.ds(c * _CH, _CH)]], rows_v, sem
        ).wait()
        pltpu.sync_copy(rows_v, out_hbm.at[pl.ds(base + c * _CH, _CH)])
        return ()

    lax.fori_loop(0, _NCHUNK, body, ())


@jax.jit
def _embed(x_flat, weights):
    mesh = plsc.VectorSubcoreMesh(core_axis_name="c", subcore_axis_name="s")
    run = functools.partial(
        pl.kernel,
        mesh=mesh,
        out_type=jax.ShapeDtypeStruct((_B, _D), jnp.float32),
        scratch_types=[
            pltpu.VMEM((_BPW,), jnp.int32),
            pltpu.VMEM((_CH, _D), jnp.float32),
            pltpu.SemaphoreType.DMA,
        ],
    )(_embed_kernel)
    return run(x_flat, weights)


def kernel(x, weights):
    out = _embed(x.reshape(-1), weights)
    return out.reshape(x.shape + (weights.shape[1],))


# speedup vs baseline: 1.8136x; 1.0738x over previous
import functools

import jax
import jax.numpy as jnp
from jax import lax
from jax.experimental import pallas as pl
from jax.experimental.pallas import tpu as pltpu
from jax.experimental.pallas import tpu_sc as plsc

_B = 16384
_D = 4096
_NW = 32
_BPW = _B // _NW
_CH = 8
_NCHUNK = _BPW // _CH


def _embed_kernel(x_hbm, table_hbm, out_hbm,
                  idx_v, rows0, rows1, gs0, gs1, os0, os1):
    nc = 2
    wid = lax.axis_index("s") * nc + lax.axis_index("c")
    base = wid * _BPW
    pltpu.sync_copy(x_hbm.at[pl.ds(base, _BPW)], idx_v)

    bufs = (rows0, rows1)
    gsem = (gs0, gs1)
    osem = (os0, os1)

    def gstart(c, b):
        pltpu.make_async_copy(
            table_hbm.at[idx_v.at[pl.ds(c * _CH, _CH)]], bufs[b], gsem[b]
        ).start()

    def gwait(b):
        pltpu.make_async_copy(
            table_hbm.at[idx_v.at[pl.ds(0, _CH)]], bufs[b], gsem[b]
        ).wait()

    def ostart(c, b):
        pltpu.make_async_copy(
            bufs[b], out_hbm.at[pl.ds(base + c * _CH, _CH)], osem[b]
        ).start()

    def owait(b):
        pltpu.make_async_copy(
            bufs[b], out_hbm.at[pl.ds(base, _CH)], osem[b]
        ).wait()

    gstart(0, 0)
    gwait(0)
    gstart(1, 1)
    ostart(0, 0)

    def body(r, _):
        c1 = 2 * r - 1
        gwait(1)
        owait(0)
        gstart(c1 + 1, 0)
        ostart(c1, 1)
        c0 = 2 * r
        gwait(0)
        owait(1)
        gstart(c0 + 1, 1)
        ostart(c0, 0)
        return ()

    lax.fori_loop(1, _NCHUNK // 2, body, ())

    gwait(1)
    owait(0)
    ostart(_NCHUNK - 1, 1)
    owait(1)


@jax.jit
def _embed(x_flat, weights):
    mesh = plsc.VectorSubcoreMesh(core_axis_name="c", subcore_axis_name="s")
    run = functools.partial(
        pl.kernel,
        mesh=mesh,
        out_type=jax.ShapeDtypeStruct((_B, _D), jnp.float32),
        scratch_types=[
            pltpu.VMEM((_BPW,), jnp.int32),
            pltpu.VMEM((_CH, _D), jnp.float32),
            pltpu.VMEM((_CH, _D), jnp.float32),
            pltpu.SemaphoreType.DMA,
            pltpu.SemaphoreType.DMA,
            pltpu.SemaphoreType.DMA,
            pltpu.SemaphoreType.DMA,
        ],
    )(_embed_kernel)
    return run(x_flat, weights)


def kernel(x, weights):
    out = _embed(x.reshape(-1), weights)
    return out.reshape(x.shape + (weights.shape[1],))


# speedup vs baseline: 1.8351x; 1.0118x over previous
import functools

import jax
import jax.numpy as jnp
from jax import lax
from jax.experimental import pallas as pl
from jax.experimental.pallas import tpu as pltpu
from jax.experimental.pallas import tpu_sc as plsc

_B = 16384
_D = 4096
_NW = 32
_BPW = _B // _NW
_CH = 8
_NCHUNK = _BPW // _CH
_NBUF = 3


def _embed_kernel(x_hbm, table_hbm, out_hbm,
                  idx_v, rows0, rows1, rows2, gs0, gs1, gs2, os0, os1, os2):
    nc = 2
    wid = lax.axis_index("s") * nc + lax.axis_index("c")
    base = wid * _BPW
    pltpu.sync_copy(x_hbm.at[pl.ds(base, _BPW)], idx_v)

    bufs = (rows0, rows1, rows2)
    gsem = (gs0, gs1, gs2)
    osem = (os0, os1, os2)

    def gstart(c, b):
        pltpu.make_async_copy(
            table_hbm.at[idx_v.at[pl.ds(c * _CH, _CH)]], bufs[b], gsem[b]
        ).start()

    def gwait(b):
        pltpu.make_async_copy(
            table_hbm.at[idx_v.at[pl.ds(0, _CH)]], bufs[b], gsem[b]
        ).wait()

    def ostart(c, b):
        pltpu.make_async_copy(
            bufs[b], out_hbm.at[pl.ds(base + c * _CH, _CH)], osem[b]
        ).start()

    def owait(b):
        pltpu.make_async_copy(
            bufs[b], out_hbm.at[pl.ds(base, _CH)], osem[b]
        ).wait()

    gstart(0, 0)
    gstart(1, 1)
    gwait(0)
    gstart(2, 2)
    ostart(0, 0)
    gwait(1)
    owait(0)
    gstart(3, 0)
    ostart(1, 1)

    def body(r, _):
        for j in range(3):
            c = 2 + 3 * r + j
            s = (2 + j) % 3
            s2 = (1 + j) % 3
            gwait(s)
            owait(s2)
            gstart(c + 2, s2)
            ostart(c, s)
        return ()

    lax.fori_loop(0, (_NCHUNK - 4) // 3, body, ())

    gwait(2)
    owait(1)
    ostart(_NCHUNK - 2, 2)
    gwait(0)
    ostart(_NCHUNK - 1, 0)
    owait(2)
    owait(0)


@jax.jit
def _embed(x_flat, weights):
    mesh = plsc.VectorSubcoreMesh(core_axis_name="c", subcore_axis_name="s")
    run = functools.partial(
        pl.kernel,
        mesh=mesh,
        out_type=jax.ShapeDtypeStruct((_B, _D), jnp.float32),
        scratch_types=[
            pltpu.VMEM((_BPW,), jnp.int32),
            pltpu.VMEM((_CH, _D), jnp.float32),
            pltpu.VMEM((_CH, _D), jnp.float32),
            pltpu.VMEM((_CH, _D), jnp.float32),
            pltpu.SemaphoreType.DMA,
            pltpu.SemaphoreType.DMA,
            pltpu.SemaphoreType.DMA,
            pltpu.SemaphoreType.DMA,
            pltpu.SemaphoreType.DMA,
            pltpu.SemaphoreType.DMA,
        ],
    )(_embed_kernel)
    return run(x_flat, weights)


def kernel(x, weights):
    out = _embed(x.reshape(-1), weights)
    return out.reshape(x.shape + (weights.shape[1],))
